# baseline (speedup 1.0000x reference)
import functools

import jax
import jax.numpy as jnp
from jax import lax
from jax.experimental import pallas as pl
from jax.experimental.pallas import tpu as pltpu
from jax.experimental.pallas import tpu_sc as plsc

N = 10000
E = 320000
D = 128
H = 128
C = 40

NP = 10240
NW = 32
CH = 128
NBUF = 2
NPH = 2
NCHUNK = 80
NCP = NCHUNK // NPH
TILE_E = NCHUNK * CH
EP = NW * TILE_E
STRIPE = NP // 16

_mesh = plsc.VectorSubcoreMesh(core_axis_name="c", subcore_axis_name="s",
                               num_cores=2, num_subcores=16)


@functools.partial(
    pl.kernel,
    out_type=jax.ShapeDtypeStruct((2, NP, D), jnp.float32),
    mesh=_mesh,
    scratch_types=[
        pltpu.VMEM((NCHUNK, CH), jnp.int32),
        pltpu.VMEM((CH, D), jnp.float32),
        pltpu.VMEM_SHARED((NP, D), jnp.float32),
    ],
)
def _sc_hist(dst_hbm, ones_hbm, zeros_hbm, out_hbm, idx_v, ones_v, acc):
    c = lax.axis_index("c")
    s = lax.axis_index("s")
    wid = s * 2 + c
    pltpu.sync_copy(zeros_hbm.at[pl.ds(s * STRIPE, STRIPE)],
                    acc.at[pl.ds(s * STRIPE, STRIPE)])
    pltpu.sync_copy(ones_hbm, ones_v)
    pltpu.sync_copy(dst_hbm.at[wid], idx_v)
    plsc.subcore_barrier()

    def body(j, carry):
        pltpu.sync_copy(ones_v, acc.at[idx_v.at[j]], add=True)
        return carry

    lax.fori_loop(0, NCHUNK, body, 0)
    plsc.subcore_barrier()
    pltpu.sync_copy(acc.at[pl.ds(s * STRIPE, STRIPE)],
                    out_hbm.at[c, pl.ds(s * STRIPE, STRIPE)])


@functools.partial(
    pl.kernel,
    out_type=jax.ShapeDtypeStruct((2, NP, D), jnp.float32),
    mesh=_mesh,
    scratch_types=[
        pltpu.VMEM((NCP, CH), jnp.int32),
        pltpu.VMEM((NCP, CH), jnp.int32),
        pltpu.VMEM((NBUF, CH, D), jnp.float32),
        pltpu.VMEM_SHARED((NP, D), jnp.float32),
        pltpu.SemaphoreType.DMA,
        pltpu.SemaphoreType.DMA,
    ],
)
def _sc_edge_scatter(y_hbm, src_hbm, dst_hbm, zeros_hbm, out_hbm,
                     src_v, dst_v, rows_v, acc, sem0, sem1):
    c = lax.axis_index("c")
    s = lax.axis_index("s")
    wid = s * 2 + c
    sems = (sem0, sem1)
    pltpu.sync_copy(zeros_hbm.at[pl.ds(s * STRIPE, STRIPE)],
                    acc.at[pl.ds(s * STRIPE, STRIPE)])
    plsc.subcore_barrier()

    for p in range(NPH):
        pltpu.sync_copy(src_hbm.at[wid, pl.ds(p * NCP, NCP)], src_v)
        pltpu.sync_copy(dst_hbm.at[wid, pl.ds(p * NCP, NCP)], dst_v)

        for b in range(NBUF):
            pltpu.async_copy(y_hbm.at[src_v.at[b]], rows_v.at[b], sems[b])

        def body(i, carry):
            j2 = i * NBUF
            for b in range(NBUF):
                j = j2 + b
                pltpu.make_async_copy(y_hbm.at[src_v.at[j]],
                                      rows_v.at[b], sems[b]).wait()
                pltpu.sync_copy(rows_v.at[b], acc.at[dst_v.at[j]], add=True)

                @pl.when(j + NBUF < NCP)
                def _():
                    pltpu.async_copy(y_hbm.at[src_v.at[j + NBUF]],
                                     rows_v.at[b], sems[b])
            return carry

        lax.fori_loop(0, NCP // NBUF, body, 0)
    plsc.subcore_barrier()
    pltpu.sync_copy(acc.at[pl.ds(s * STRIPE, STRIPE)],
                    out_hbm.at[c, pl.ds(s * STRIPE, STRIPE)])


BM = 512


def _dinv_from_hist(hist_blk):
    deg = hist_blk[0, :, 0:1] + hist_blk[1, :, 0:1] + 1.0
    return lax.rsqrt(deg)


def _mm1_body(x_ref, w_ref, o_ref):
    o_ref[...] = jnp.dot(x_ref[...], w_ref[...],
                         preferred_element_type=jnp.float32)


def _scale_body(hist_ref, g_ref, o_ref):
    o_ref[...] = g_ref[...] * _dinv_from_hist(hist_ref[...])


def _mm2_body(hist_ref, p_ref, y_ref, b_ref, w_ref, o_ref):
    dinv = _dinv_from_hist(hist_ref[...])
    p = p_ref[...]
    a = jnp.maximum((p[0] + p[1] + y_ref[...]) * dinv + b_ref[...], 0.0)
    o_ref[...] = jnp.dot(a, w_ref[...],
                         preferred_element_type=jnp.float32) * dinv


def _mm3_body(hist_ref, q_ref, y_ref, b_ref, wl_ref, bl_ref, h_ref, z_ref):
    dinv = _dinv_from_hist(hist_ref[...])
    q = q_ref[...]
    h = jnp.maximum((q[0] + q[1] + y_ref[...]) * dinv + b_ref[...], 0.0)
    h_ref[...] = h
    z_ref[...] = jnp.dot(h, wl_ref[...],
                         preferred_element_type=jnp.float32) + bl_ref[...]


_hist_spec = pl.BlockSpec((2, BM, D), lambda i: (0, i, 0))
_row_spec = pl.BlockSpec((BM, D), lambda i: (i, 0))
_pair_spec = pl.BlockSpec((2, BM, D), lambda i: (0, i, 0))
_w_spec = pl.BlockSpec((D, D), lambda i: (0, 0))
_b_spec = pl.BlockSpec((1, D), lambda i: (0, 0))
_grid = (NP // BM,)

_mm1 = pl.pallas_call(
    _mm1_body,
    grid=_grid,
    in_specs=[_row_spec, _w_spec],
    out_specs=_row_spec,
    out_shape=jax.ShapeDtypeStruct((NP, D), jnp.float32),
)

_scale = pl.pallas_call(
    _scale_body,
    grid=_grid,
    in_specs=[_hist_spec, _row_spec],
    out_specs=_row_spec,
    out_shape=jax.ShapeDtypeStruct((NP, D), jnp.float32),
)

_mm2 = pl.pallas_call(
    _mm2_body,
    grid=_grid,
    in_specs=[_hist_spec, _pair_spec, _row_spec, _b_spec, _w_spec],
    out_specs=_row_spec,
    out_shape=jax.ShapeDtypeStruct((NP, D), jnp.float32),
)

_mm3 = pl.pallas_call(
    _mm3_body,
    grid=_grid,
    in_specs=[_hist_spec, _pair_spec, _row_spec, _b_spec,
              pl.BlockSpec((D, C), lambda i: (0, 0)),
              pl.BlockSpec((1, C), lambda i: (0, 0))],
    out_specs=[_row_spec, pl.BlockSpec((BM, C), lambda i: (i, 0))],
    out_shape=[jax.ShapeDtypeStruct((NP, D), jnp.float32),
               jax.ShapeDtypeStruct((NP, C), jnp.float32)],
)


def kernel(x, edge_index, W1, b1, W2, b2, Wl, bl):
    src = edge_index[0]
    dst = edge_index[1]
    npad = EP - E
    pad_idx = (N + jnp.arange(npad, dtype=jnp.int32) % (NP - N))
    src_p = jnp.concatenate([src, pad_idx]).reshape(NW, NCHUNK, CH)
    dst_p = jnp.concatenate([dst, pad_idx]).reshape(NW, NCHUNK, CH)

    x_p = jnp.zeros((NP, D), jnp.float32).at[:N].set(x)
    zeros_nd = jnp.zeros((NP, D), jnp.float32)
    ones_ch = jnp.ones((CH, D), jnp.float32)

    hist = _sc_hist(dst_p, ones_ch, zeros_nd)

    g1 = _mm1(x_p, W1)
    y1 = _scale(hist, g1)
    p1 = _sc_edge_scatter(y1, src_p, dst_p, zeros_nd)
    y2 = _mm2(hist, p1, y1, b1.reshape(1, H), W2)
    p2 = _sc_edge_scatter(y2, src_p, dst_p, zeros_nd)
    h_full, z_full = _mm3(hist, p2, y2, b2.reshape(1, H), Wl,
                          bl.reshape(1, C))

    return z_full[:N], h_full[:N]


# speedup vs baseline: 1.0052x; 1.0052x over previous
import functools

import jax
import jax.numpy as jnp
from jax import lax
from jax.experimental import pallas as pl
from jax.experimental.pallas import tpu as pltpu
from jax.experimental.pallas import tpu_sc as plsc

N = 10000
E = 320000
D = 128
H = 128
C = 40

NP = 10240
NW = 32
CH = 128
NBUF = 2
NPH = 2
NCHUNK = 80
NCP = NCHUNK // NPH
TILE_E = NCHUNK * CH
EP = NW * TILE_E
STRIPE = NP // 16

_mesh = plsc.VectorSubcoreMesh(core_axis_name="c", subcore_axis_name="s",
                               num_cores=2, num_subcores=16)


@functools.partial(
    pl.kernel,
    out_type=jax.ShapeDtypeStruct((2, NP, D), jnp.float32),
    mesh=_mesh,
    scratch_types=[
        pltpu.VMEM((NCHUNK, CH), jnp.int32),
        pltpu.VMEM((CH, D), jnp.float32),
        pltpu.VMEM_SHARED((NP, D), jnp.float32),
    ],
)
def _sc_hist(dst_hbm, ones_hbm, zeros_hbm, out_hbm, idx_v, ones_v, acc):
    c = lax.axis_index("c")
    s = lax.axis_index("s")
    wid = s * 2 + c
    pltpu.sync_copy(zeros_hbm.at[pl.ds(s * STRIPE, STRIPE)],
                    acc.at[pl.ds(s * STRIPE, STRIPE)])
    pltpu.sync_copy(ones_hbm, ones_v)
    pltpu.sync_copy(dst_hbm.at[wid], idx_v)
    plsc.subcore_barrier()

    def body(j, carry):
        pltpu.sync_copy(ones_v, acc.at[idx_v.at[j]], add=True)
        return carry

    lax.fori_loop(0, NCHUNK, body, 0)
    plsc.subcore_barrier()
    pltpu.sync_copy(acc.at[pl.ds(s * STRIPE, STRIPE)],
                    out_hbm.at[c, pl.ds(s * STRIPE, STRIPE)])


@functools.partial(
    pl.kernel,
    out_type=jax.ShapeDtypeStruct((2, NP, D), jnp.float32),
    mesh=_mesh,
    scratch_types=[
        pltpu.VMEM((NCP, CH), jnp.int32),
        pltpu.VMEM((NCP, CH), jnp.int32),
        pltpu.VMEM((NBUF, CH, D), jnp.float32),
        pltpu.VMEM_SHARED((NP, D), jnp.float32),
        pltpu.SemaphoreType.DMA,
        pltpu.SemaphoreType.DMA,
    ],
)
def _sc_edge_scatter(y_hbm, src_hbm, dst_hbm, zeros_hbm, out_hbm,
                     src_v, dst_v, rows_v, acc, sem0, sem1):
    c = lax.axis_index("c")
    s = lax.axis_index("s")
    wid = s * 2 + c
    sems = (sem0, sem1)
    pltpu.sync_copy(zeros_hbm.at[pl.ds(s * STRIPE, STRIPE)],
                    acc.at[pl.ds(s * STRIPE, STRIPE)])
    plsc.subcore_barrier()

    for p in range(NPH):
        pltpu.sync_copy(src_hbm.at[wid, pl.ds(p * NCP, NCP)], src_v)
        pltpu.sync_copy(dst_hbm.at[wid, pl.ds(p * NCP, NCP)], dst_v)

        for b in range(NBUF):
            pltpu.async_copy(y_hbm.at[src_v.at[b]], rows_v.at[b], sems[b])

        def body(i, carry):
            j2 = i * NBUF
            for b in range(NBUF):
                j = j2 + b
                pltpu.make_async_copy(y_hbm.at[src_v.at[j]],
                                      rows_v.at[b], sems[b]).wait()
                pltpu.sync_copy(rows_v.at[b], acc.at[dst_v.at[j]], add=True)

                @pl.when(j + NBUF < NCP)
                def _():
                    pltpu.async_copy(y_hbm.at[src_v.at[j + NBUF]],
                                     rows_v.at[b], sems[b])
            return carry

        lax.fori_loop(0, NCP // NBUF, body, 0)
    plsc.subcore_barrier()
    pltpu.sync_copy(acc.at[pl.ds(s * STRIPE, STRIPE)],
                    out_hbm.at[c, pl.ds(s * STRIPE, STRIPE)])


BM = 512


def _dinv_from_hist(hist_blk):
    deg = hist_blk[0, :, 0:1] + hist_blk[1, :, 0:1] + 1.0
    return lax.rsqrt(deg)


def _mm1_body(hist_ref, x_ref, w_ref, o_ref):
    dinv = _dinv_from_hist(hist_ref[...])
    o_ref[...] = jnp.dot(x_ref[...], w_ref[...],
                         preferred_element_type=jnp.float32) * dinv


def _mm2_body(hist_ref, p_ref, y_ref, b_ref, w_ref, o_ref):
    dinv = _dinv_from_hist(hist_ref[...])
    p = p_ref[...]
    a = jnp.maximum((p[0] + p[1] + y_ref[...]) * dinv + b_ref[...], 0.0)
    o_ref[...] = jnp.dot(a, w_ref[...],
                         preferred_element_type=jnp.float32) * dinv


def _mm3_body(hist_ref, q_ref, y_ref, b_ref, wl_ref, bl_ref, h_ref, z_ref):
    dinv = _dinv_from_hist(hist_ref[...])
    q = q_ref[...]
    h = jnp.maximum((q[0] + q[1] + y_ref[...]) * dinv + b_ref[...], 0.0)
    h_ref[...] = h
    z_ref[...] = jnp.dot(h, wl_ref[...],
                         preferred_element_type=jnp.float32) + bl_ref[...]


_hist_spec = pl.BlockSpec((2, BM, D), lambda i: (0, i, 0))
_row_spec = pl.BlockSpec((BM, D), lambda i: (i, 0))
_pair_spec = pl.BlockSpec((2, BM, D), lambda i: (0, i, 0))
_w_spec = pl.BlockSpec((D, D), lambda i: (0, 0))
_b_spec = pl.BlockSpec((1, D), lambda i: (0, 0))
_grid = (NP // BM,)

_mm1 = pl.pallas_call(
    _mm1_body,
    grid=_grid,
    in_specs=[_hist_spec, _row_spec, _w_spec],
    out_specs=_row_spec,
    out_shape=jax.ShapeDtypeStruct((NP, D), jnp.float32),
)

_mm2 = pl.pallas_call(
    _mm2_body,
    grid=_grid,
    in_specs=[_hist_spec, _pair_spec, _row_spec, _b_spec, _w_spec],
    out_specs=_row_spec,
    out_shape=jax.ShapeDtypeStruct((NP, D), jnp.float32),
)

_mm3 = pl.pallas_call(
    _mm3_body,
    grid=_grid,
    in_specs=[_hist_spec, _pair_spec, _row_spec, _b_spec,
              pl.BlockSpec((D, C), lambda i: (0, 0)),
              pl.BlockSpec((1, C), lambda i: (0, 0))],
    out_specs=[_row_spec, pl.BlockSpec((BM, C), lambda i: (i, 0))],
    out_shape=[jax.ShapeDtypeStruct((NP, D), jnp.float32),
               jax.ShapeDtypeStruct((NP, C), jnp.float32)],
)


def kernel(x, edge_index, W1, b1, W2, b2, Wl, bl):
    src = edge_index[0]
    dst = edge_index[1]
    npad = EP - E
    pad_idx = (N + jnp.arange(npad, dtype=jnp.int32) % (NP - N))
    src_p = jnp.concatenate([src, pad_idx]).reshape(NW, NCHUNK, CH)
    dst_p = jnp.concatenate([dst, pad_idx]).reshape(NW, NCHUNK, CH)

    x_p = jnp.zeros((NP, D), jnp.float32).at[:N].set(x)
    zeros_nd = jnp.zeros((NP, D), jnp.float32)
    ones_ch = jnp.ones((CH, D), jnp.float32)

    hist = _sc_hist(dst_p, ones_ch, zeros_nd)

    y1 = _mm1(hist, x_p, W1)
    p1 = _sc_edge_scatter(y1, src_p, dst_p, zeros_nd)
    y2 = _mm2(hist, p1, y1, b1.reshape(1, H), W2)
    p2 = _sc_edge_scatter(y2, src_p, dst_p, zeros_nd)
    h_full, z_full = _mm3(hist, p2, y2, b2.reshape(1, H), Wl,
                          bl.reshape(1, C))

    return z_full[:N], h_full[:N]
